# baseline (speedup 1.0000x reference)
import functools

import jax
import jax.numpy as jnp
from jax.experimental import pallas as pl
from jax.experimental.pallas import tpu as pltpu

_B, _T, _DA, _DT, _V = 16, 8, 768, 512, 49408
_N = _B * _T
_TAU = 0.1
_VB = 1024
_NB = (_V + _VB - 1) // _VB


def _stats_kernel(emb_ref, audio_ref, w_ref, b_ref, fn_ref, sum_ref, sq_ref):
    i = pl.program_id(0)

    @pl.when(i == 0)
    def _init():
        sum_ref[...] = jnp.zeros_like(sum_ref)
        sq_ref[...] = jnp.zeros_like(sq_ref)

    e = emb_ref[...]
    base = i * _VB
    row_ok = (jax.lax.broadcasted_iota(jnp.int32, (_VB, 1), 0) + base) < _V
    e = jnp.where(row_ok, e, 0.0)
    sum_ref[...] += jnp.sum(e, axis=0, keepdims=True)
    sq_ref[...] += jnp.sum(e * e, axis=0, keepdims=True)

    @pl.when(i == _NB - 1)
    def _finish():
        emb_mean = sum_ref[...] / _V
        emb_var = sq_ref[...] / _V - emb_mean * emb_mean
        emb_std = jnp.sqrt(jnp.maximum(emb_var, 0.0))
        feats = (
            jax.lax.dot_general(
                audio_ref[...], w_ref[...], (((1,), (0,)), ((), ())),
                preferred_element_type=jnp.float32,
                precision=jax.lax.Precision.HIGHEST,
            )
            + b_ref[...]
        )
        mu = jnp.mean(feats, axis=0, keepdims=True)
        var = jnp.mean((feats - mu) * (feats - mu), axis=0, keepdims=True)
        normed = (feats - mu) * jax.lax.rsqrt(var + 1e-5)
        f = normed * emb_std + emb_mean
        norm = jnp.sqrt(jnp.sum(f * f, axis=1, keepdims=True)) + 1e-8
        fn_ref[...] = f / norm


def _score_kernel(fn_ref, emb_ref, cos_ref, kw_ref, acc_ref, l_ref):
    i = pl.program_id(0)

    @pl.when(i == 0)
    def _init():
        acc_ref[...] = jnp.zeros_like(acc_ref)
        l_ref[...] = jnp.zeros_like(l_ref)

    e = emb_ref[...]
    base = i * _VB
    row_ok = (jax.lax.broadcasted_iota(jnp.int32, (_VB, 1), 0) + base) < _V
    e = jnp.where(row_ok, e, 0.0)
    inv_norm = 1.0 / (jnp.sqrt(jnp.sum(e * e, axis=1, keepdims=True)) + 1e-8)
    e_n = e * inv_norm
    s = jax.lax.dot_general(
        fn_ref[...], e_n, (((1,), (1,)), ((), ())),
        preferred_element_type=jnp.float32,
        precision=jax.lax.Precision.HIGHEST,
    )
    cos_ref[...] = s
    p = jnp.exp(s * (1.0 / _TAU))
    col_ok = jax.lax.broadcasted_iota(jnp.int32, (1, _VB), 1) < (_V - base)
    p = jnp.where(col_ok, p, 0.0)
    l_ref[...] += jnp.sum(p, axis=1, keepdims=True)
    acc_ref[...] += jax.lax.dot_general(
        p, e, (((1,), (0,)), ((), ())),
        preferred_element_type=jnp.float32,
        precision=jax.lax.Precision.HIGHEST,
    )

    @pl.when(i == _NB - 1)
    def _finish():
        kw_ref[...] = acc_ref[...] / l_ref[...]


@functools.partial(jax.jit, static_argnames=("interpret",))
def _run(audio_feat, W_proj, b_proj, token_embedding, interpret=False):
    audio2d = audio_feat.reshape(_N, _DA)
    b2d = b_proj.reshape(1, _DT)

    fn = pl.pallas_call(
        _stats_kernel,
        grid=(_NB,),
        in_specs=[
            pl.BlockSpec((_VB, _DT), lambda i: (i, 0)),
            pl.BlockSpec((_N, _DA), lambda i: (0, 0)),
            pl.BlockSpec((_DA, _DT), lambda i: (0, 0)),
            pl.BlockSpec((1, _DT), lambda i: (0, 0)),
        ],
        out_specs=pl.BlockSpec((_N, _DT), lambda i: (0, 0)),
        out_shape=jax.ShapeDtypeStruct((_N, _DT), jnp.float32),
        scratch_shapes=[
            pltpu.VMEM((1, _DT), jnp.float32),
            pltpu.VMEM((1, _DT), jnp.float32),
        ],
        compiler_params=pltpu.CompilerParams(
            dimension_semantics=("arbitrary",),
        ),
        interpret=interpret,
    )(token_embedding, audio2d, W_proj, b2d)

    cos, kw = pl.pallas_call(
        _score_kernel,
        grid=(_NB,),
        in_specs=[
            pl.BlockSpec((_N, _DT), lambda i: (0, 0)),
            pl.BlockSpec((_VB, _DT), lambda i: (i, 0)),
        ],
        out_specs=[
            pl.BlockSpec((_N, _VB), lambda i: (0, i)),
            pl.BlockSpec((_N, _DT), lambda i: (0, 0)),
        ],
        out_shape=[
            jax.ShapeDtypeStruct((_N, _V), jnp.float32),
            jax.ShapeDtypeStruct((_N, _DT), jnp.float32),
        ],
        scratch_shapes=[
            pltpu.VMEM((_N, _DT), jnp.float32),
            pltpu.VMEM((_N, 1), jnp.float32),
        ],
        compiler_params=pltpu.CompilerParams(
            dimension_semantics=("arbitrary",),
        ),
        interpret=interpret,
    )(fn, token_embedding)

    keywords = kw.reshape(_B, _T, _DT)
    cos_score = cos.reshape(_B, _T, _V)
    return keywords, cos_score


def kernel(audio_feat, W_proj, b_proj, token_embedding):
    return _run(audio_feat, W_proj, b_proj, token_embedding)


# speedup vs baseline: 1.8815x; 1.8815x over previous
import functools

import jax
import jax.numpy as jnp
from jax.experimental import pallas as pl
from jax.experimental.pallas import tpu as pltpu

_B, _T, _DA, _DT, _V = 16, 8, 768, 512, 49408
_N = _B * _T
_TAU = 0.1
_VB = 1024
_NB = (_V + _VB - 1) // _VB


def _stats_kernel(emb_ref, audio_ref, w_ref, b_ref, fn_ref, sum_ref, sq_ref):
    i = pl.program_id(0)

    @pl.when(i == 0)
    def _init():
        sum_ref[...] = jnp.zeros_like(sum_ref)
        sq_ref[...] = jnp.zeros_like(sq_ref)

    e = emb_ref[...]
    base = i * _VB
    row_ok = (jax.lax.broadcasted_iota(jnp.int32, (_VB, 1), 0) + base) < _V
    e = jnp.where(row_ok, e, 0.0)
    sum_ref[...] += jnp.sum(e, axis=0, keepdims=True)
    sq_ref[...] += jnp.sum(e * e, axis=0, keepdims=True)

    @pl.when(i == _NB - 1)
    def _finish():
        emb_mean = sum_ref[...] / _V
        emb_var = sq_ref[...] / _V - emb_mean * emb_mean
        emb_std = jnp.sqrt(jnp.maximum(emb_var, 0.0))
        feats = (
            jax.lax.dot_general(
                audio_ref[...], w_ref[...], (((1,), (0,)), ((), ())),
                preferred_element_type=jnp.float32,
                precision=jax.lax.Precision.HIGHEST,
            )
            + b_ref[...]
        )
        mu = jnp.mean(feats, axis=0, keepdims=True)
        var = jnp.mean((feats - mu) * (feats - mu), axis=0, keepdims=True)
        normed = (feats - mu) * jax.lax.rsqrt(var + 1e-5)
        f = normed * emb_std + emb_mean
        norm = jnp.sqrt(jnp.sum(f * f, axis=1, keepdims=True)) + 1e-8
        fn_ref[...] = f / norm


def _score_kernel(fn_ref, emb_ref, cos_ref, kw_ref, acc_ref, l_ref):
    i = pl.program_id(0)

    @pl.when(i == 0)
    def _init():
        acc_ref[...] = jnp.zeros_like(acc_ref)
        l_ref[...] = jnp.zeros_like(l_ref)

    e = emb_ref[...]
    base = i * _VB
    row_ok = (jax.lax.broadcasted_iota(jnp.int32, (_VB, 1), 0) + base) < _V
    e = jnp.where(row_ok, e, 0.0)
    inv_norm = 1.0 / (jnp.sqrt(jnp.sum(e * e, axis=1, keepdims=True)) + 1e-8)
    e_n = e * inv_norm
    s = jax.lax.dot_general(
        fn_ref[...], e_n, (((1,), (1,)), ((), ())),
        preferred_element_type=jnp.float32,
    )
    cos_ref[...] = s
    p = jnp.exp(s * (1.0 / _TAU))
    col_ok = jax.lax.broadcasted_iota(jnp.int32, (1, _VB), 1) < (_V - base)
    p = jnp.where(col_ok, p, 0.0)
    l_ref[...] += jnp.sum(p, axis=1, keepdims=True)
    acc_ref[...] += jax.lax.dot_general(
        p, e, (((1,), (0,)), ((), ())),
        preferred_element_type=jnp.float32,
    )

    @pl.when(i == _NB - 1)
    def _finish():
        kw_ref[...] = acc_ref[...] / l_ref[...]


@functools.partial(jax.jit, static_argnames=("interpret",))
def _run(audio_feat, W_proj, b_proj, token_embedding, interpret=False):
    audio2d = audio_feat.reshape(_N, _DA)
    b2d = b_proj.reshape(1, _DT)

    fn = pl.pallas_call(
        _stats_kernel,
        grid=(_NB,),
        in_specs=[
            pl.BlockSpec((_VB, _DT), lambda i: (i, 0)),
            pl.BlockSpec((_N, _DA), lambda i: (0, 0)),
            pl.BlockSpec((_DA, _DT), lambda i: (0, 0)),
            pl.BlockSpec((1, _DT), lambda i: (0, 0)),
        ],
        out_specs=pl.BlockSpec((_N, _DT), lambda i: (0, 0)),
        out_shape=jax.ShapeDtypeStruct((_N, _DT), jnp.float32),
        scratch_shapes=[
            pltpu.VMEM((1, _DT), jnp.float32),
            pltpu.VMEM((1, _DT), jnp.float32),
        ],
        compiler_params=pltpu.CompilerParams(
            dimension_semantics=("arbitrary",),
        ),
        interpret=interpret,
    )(token_embedding, audio2d, W_proj, b2d)

    cos, kw = pl.pallas_call(
        _score_kernel,
        grid=(_NB,),
        in_specs=[
            pl.BlockSpec((_N, _DT), lambda i: (0, 0)),
            pl.BlockSpec((_VB, _DT), lambda i: (i, 0)),
        ],
        out_specs=[
            pl.BlockSpec((_N, _VB), lambda i: (0, i)),
            pl.BlockSpec((_N, _DT), lambda i: (0, 0)),
        ],
        out_shape=[
            jax.ShapeDtypeStruct((_N, _V), jnp.float32),
            jax.ShapeDtypeStruct((_N, _DT), jnp.float32),
        ],
        scratch_shapes=[
            pltpu.VMEM((_N, _DT), jnp.float32),
            pltpu.VMEM((_N, 1), jnp.float32),
        ],
        compiler_params=pltpu.CompilerParams(
            dimension_semantics=("arbitrary",),
        ),
        interpret=interpret,
    )(fn, token_embedding)

    keywords = kw.reshape(_B, _T, _DT)
    cos_score = cos.reshape(_B, _T, _V)
    return keywords, cos_score


def kernel(audio_feat, W_proj, b_proj, token_embedding):
    return _run(audio_feat, W_proj, b_proj, token_embedding)
